# baseline (speedup 1.0000x reference)
import functools

import jax
import jax.numpy as jnp
from jax.experimental import pallas as pl
from jax.experimental.pallas import tpu as pltpu

_EPS = 1e-5
_SLOPE = 0.01
_F32 = jnp.float32
_BF16 = jnp.bfloat16


def _leaky(h):
    return jnp.maximum(h, _SLOPE * h)


def _scale_shift(s_ref, q_ref, gamma, beta, count):
    s = jnp.sum(s_ref[...], axis=0)
    q = jnp.sum(q_ref[...], axis=0)
    mean = s / count
    var = jnp.maximum(q / count - mean * mean, 0.0)
    scale = gamma * jax.lax.rsqrt(var + _EPS)
    shift = beta - mean * scale
    return scale, shift


def _p1_kernel(x_ref, gram_ref, sx_ref, *, spt, L, cin):
    g = jnp.zeros((cin, cin), _F32)
    sx = jnp.zeros((cin, 1), _F32)
    for i in range(spt):
        xb = x_ref[i].astype(_BF16)
        g = g + jax.lax.dot_general(xb, xb, (((1,), (1,)), ((), ())),
                                    preferred_element_type=_F32)
        sx = sx + jnp.sum(xb.astype(_F32), axis=1, keepdims=True)
    gram_ref[0] = g
    sx_ref[0] = jnp.transpose(sx)


def _gram_stats(gram_ref, sx_ref, w, gamma, beta, count):
    g = jnp.sum(gram_ref[...], axis=0)
    g_hi = g.astype(_BF16)
    g_lo = (g - g_hi.astype(_F32)).astype(_BF16)
    wb = w.astype(_BF16)
    gw = (jnp.dot(g_hi, wb, preferred_element_type=_F32)
          + jnp.dot(g_lo, wb, preferred_element_type=_F32))
    q = jnp.sum(w * gw, axis=0, keepdims=True)
    st = jnp.transpose(jnp.sum(sx_ref[...], axis=0))
    s = jnp.sum(st * w, axis=0, keepdims=True)
    mean = s / count
    var = jnp.maximum(q / count - mean * mean, 0.0)
    scale = gamma * jax.lax.rsqrt(var + _EPS)
    shift = beta - mean * scale
    return scale, shift


def _p2_kernel(x_ref, w1_ref, gram1_ref, sx_ref, g1_ref, be1_ref, w2_ref,
               xr_ref, h2_ref, s_ref, q_ref, *, spt, L, cmid, count):
    sc1, sh1 = _gram_stats(gram1_ref, sx_ref, w1_ref[...],
                           g1_ref[...], be1_ref[...], count)
    w = w2_ref[...].reshape(3 * cmid, cmid).astype(_BF16)
    w1s = (w1_ref[...] * sc1).astype(_BF16)
    for i in range(spt):
        xr_ref[i * L:(i + 1) * L, :] = jnp.transpose(x_ref[i].astype(_BF16))
    y1 = jnp.dot(xr_ref[...], w1s,
                 preferred_element_type=_F32)
    h = _leaky(y1 + sh1)
    hb = h.astype(_BF16)
    tm, c = hb.shape
    z = jnp.zeros((1, c), _BF16)
    left = jnp.concatenate([z, hb[:tm - 1, :]], axis=0)
    right = jnp.concatenate([hb[1:, :], z], axis=0)
    pos = jax.lax.broadcasted_iota(jnp.int32, (tm, 1), 0) % L
    left = jnp.where(pos == 0, z, left)
    right = jnp.where(pos == L - 1, z, right)
    hin = jnp.concatenate([left, hb, right], axis=1)
    y = jnp.dot(hin, w, preferred_element_type=_F32)
    h2_ref[...] = y.astype(_BF16)
    s_ref[0] = jnp.sum(y, axis=0, keepdims=True)
    q_ref[0] = jnp.sum(y * y, axis=0, keepdims=True)


def _p3_kernel(h2_ref, s2_ref, q2_ref, g2_ref, be2_ref,
               gram_ref, s_ref, *, count):
    sc2, sh2 = _scale_shift(s2_ref, q2_ref, g2_ref[...], be2_ref[...], count)
    h = _leaky(h2_ref[...].astype(_F32) * sc2 + sh2)
    hb = h.astype(_BF16)
    gram_ref[0] = jax.lax.dot_general(hb, hb, (((0,), (0,)), ((), ())),
                                      preferred_element_type=_F32)
    s_ref[0] = jnp.sum(hb.astype(_F32), axis=0, keepdims=True)


def _p4_kernel(h2_ref, xr_ref, s2_ref, q2_ref, g2_ref, be2_ref,
               gram1_ref, sx_ref, gram3_ref, s3_ref,
               gid_ref, beid_ref, g3_ref, be3_ref,
               w3_ref, wid_ref, o_ref, *, spt, L, cmid, count):
    sc2, sh2 = _scale_shift(s2_ref, q2_ref, g2_ref[...], be2_ref[...], count)
    scid, shid = _gram_stats(gram1_ref, sx_ref, wid_ref[...],
                             gid_ref[...], beid_ref[...], count)
    sc3, sh3 = _gram_stats(gram3_ref, s3_ref, w3_ref[...],
                           g3_ref[...], be3_ref[...], count)
    c4 = w3_ref.shape[1]
    zrows = jnp.zeros((cmid - 1, c4), _BF16)
    w3a = jnp.concatenate(
        [(w3_ref[...] * sc3).astype(_BF16), sh3.astype(_BF16), zrows], axis=0)
    wida = jnp.concatenate(
        [(wid_ref[...] * scid).astype(_BF16), shid.astype(_BF16), zrows], axis=0)
    ones = jnp.ones((L, cmid), _BF16)
    for i in range(spt):
        hbi = _leaky(h2_ref[i * L:(i + 1) * L, :].astype(_F32) * sc2
                     + sh2).astype(_BF16)
        hx = jnp.concatenate([hbi, ones], axis=1)
        a = _leaky(jnp.dot(hx, w3a, preferred_element_type=_F32))
        xo = jnp.concatenate([xr_ref[i * L:(i + 1) * L, :], ones], axis=1)
        idn = jnp.dot(xo, wida, preferred_element_type=_F32)
        o = _leaky(a + idn)
        o_ref[i] = jnp.transpose(o)


def kernel(x_ncl, w1, b1, g1, be1, w2, b2, g2, be2, w3, b3, g3, be3,
           wid, bid, gid, beid):
    N, Cin, L = x_ncl.shape
    Cmid = w1.shape[1]
    C4 = w3.shape[1]
    NL = N * L

    spt = 128
    while N % spt:
        spt //= 2
    T = N // spt
    TM = spt * L

    cparams = pltpu.CompilerParams(
        dimension_semantics=("parallel",),
        vmem_limit_bytes=64 * 1024 * 1024,
    )
    row_spec = lambda c: pl.BlockSpec((TM, c), lambda i: (i, 0))
    vec_spec = lambda c: pl.BlockSpec((1, c), lambda i: (0, 0))
    full_spec = lambda r, c: pl.BlockSpec((r, c), lambda i: (0, 0))
    ncl_spec = lambda c: pl.BlockSpec((spt, c, L), lambda i: (i, 0, 0))
    stat_spec = lambda c: pl.BlockSpec((1, 1, c), lambda i: (i, 0, 0))
    allstat_spec = lambda c: pl.BlockSpec((T, 1, c), lambda i: (0, 0, 0))
    stat_shape = lambda c: jax.ShapeDtypeStruct((T, 1, c), _F32)
    slab_shape = lambda c: jax.ShapeDtypeStruct((NL, c), _BF16)
    w2_spec = pl.BlockSpec((3, Cmid, Cmid), lambda i: (0, 0, 0))

    gram_out_spec = pl.BlockSpec((1, Cin, Cin), lambda i: (i, 0, 0))
    gram_in_spec = pl.BlockSpec((T, Cin, Cin), lambda i: (0, 0, 0))
    gram_shape = jax.ShapeDtypeStruct((T, Cin, Cin), _F32)

    gram1, sx = pl.pallas_call(
        functools.partial(_p1_kernel, spt=spt, L=L, cin=Cin),
        grid=(T,),
        in_specs=[ncl_spec(Cin)],
        out_specs=[gram_out_spec, stat_spec(Cin)],
        out_shape=[gram_shape, stat_shape(Cin)],
        compiler_params=cparams,
    )(x_ncl)

    xr, h2, s2, q2 = pl.pallas_call(
        functools.partial(_p2_kernel, spt=spt, L=L, cmid=Cmid, count=float(NL)),
        grid=(T,),
        in_specs=[ncl_spec(Cin), full_spec(Cin, Cmid),
                  gram_in_spec, allstat_spec(Cin),
                  vec_spec(Cmid), vec_spec(Cmid), w2_spec],
        out_specs=[row_spec(Cin), row_spec(Cmid),
                   stat_spec(Cmid), stat_spec(Cmid)],
        out_shape=[slab_shape(Cin), slab_shape(Cmid),
                   stat_shape(Cmid), stat_shape(Cmid)],
        compiler_params=cparams,
    )(x_ncl, w1, gram1, sx, g1, be1, w2)

    gram3, s3 = pl.pallas_call(
        functools.partial(_p3_kernel, count=float(NL)),
        grid=(T,),
        in_specs=[row_spec(Cmid), allstat_spec(Cmid), allstat_spec(Cmid),
                  vec_spec(Cmid), vec_spec(Cmid)],
        out_specs=[pl.BlockSpec((1, Cmid, Cmid), lambda i: (i, 0, 0)),
                   stat_spec(Cmid)],
        out_shape=[jax.ShapeDtypeStruct((T, Cmid, Cmid), _F32),
                   stat_shape(Cmid)],
        compiler_params=cparams,
    )(h2, s2, q2, g2, be2)

    spt4 = min(spt, 64)
    T4 = N // spt4
    TM4 = spt4 * L
    row4_spec = lambda c: pl.BlockSpec((TM4, c), lambda i: (i, 0))
    out = pl.pallas_call(
        functools.partial(_p4_kernel, spt=spt4, L=L, cmid=Cmid, count=float(NL)),
        grid=(T4,),
        in_specs=[row4_spec(Cmid), row4_spec(Cin),
                  allstat_spec(Cmid), allstat_spec(Cmid),
                  vec_spec(Cmid), vec_spec(Cmid),
                  gram_in_spec, allstat_spec(Cin),
                  pl.BlockSpec((T, Cmid, Cmid), lambda i: (0, 0, 0)),
                  allstat_spec(Cmid),
                  vec_spec(C4), vec_spec(C4),
                  vec_spec(C4), vec_spec(C4),
                  full_spec(Cmid, C4), full_spec(Cin, C4)],
        out_specs=pl.BlockSpec((spt4, C4, L), lambda i: (i, 0, 0)),
        out_shape=jax.ShapeDtypeStruct((N, C4, L), _F32),
        compiler_params=cparams,
    )(h2, xr, s2, q2, g2, be2, gram1, sx, gram3, s3,
      gid, beid, g3, be3, w3, wid)
    return out


# speedup vs baseline: 1.0209x; 1.0209x over previous
import functools

import jax
import jax.numpy as jnp
from jax.experimental import pallas as pl
from jax.experimental.pallas import tpu as pltpu

_EPS = 1e-5
_SLOPE = 0.01
_F32 = jnp.float32
_BF16 = jnp.bfloat16


def _leaky(h):
    return jnp.maximum(h, _SLOPE * h)


def _scale_shift(s_ref, q_ref, gamma, beta, count):
    s = jnp.sum(s_ref[...], axis=0)
    q = jnp.sum(q_ref[...], axis=0)
    mean = s / count
    var = jnp.maximum(q / count - mean * mean, 0.0)
    scale = gamma * jax.lax.rsqrt(var + _EPS)
    shift = beta - mean * scale
    return scale, shift


def _p1_kernel(x_ref, xr_ref, gram_ref, sx_ref, *, spt, L):
    for i in range(spt):
        xt = jnp.transpose(x_ref[i].astype(_BF16))
        xr_ref[i * L:(i + 1) * L, :] = xt
    xall = xr_ref[...]
    gram_ref[0] = jax.lax.dot_general(xall, xall, (((0,), (0,)), ((), ())),
                                      preferred_element_type=_F32)
    sx_ref[0] = jnp.sum(xall.astype(_F32), axis=0, keepdims=True)


def _gram_stats(gram_ref, sx_ref, w, gamma, beta, count):
    g = jnp.sum(gram_ref[...], axis=0)
    g_hi = g.astype(_BF16)
    g_lo = (g - g_hi.astype(_F32)).astype(_BF16)
    wb = w.astype(_BF16)
    gw = (jnp.dot(g_hi, wb, preferred_element_type=_F32)
          + jnp.dot(g_lo, wb, preferred_element_type=_F32))
    q = jnp.sum(w * gw, axis=0, keepdims=True)
    st = jnp.transpose(jnp.sum(sx_ref[...], axis=0))
    s = jnp.sum(st * w, axis=0, keepdims=True)
    mean = s / count
    var = jnp.maximum(q / count - mean * mean, 0.0)
    scale = gamma * jax.lax.rsqrt(var + _EPS)
    shift = beta - mean * scale
    return scale, shift


def _p2_kernel(xr_ref, w1_ref, gram1_ref, sx_ref, g1_ref, be1_ref, w2_ref,
               h2_ref, s_ref, q_ref, *, L, cmid, count):
    sc1, sh1 = _gram_stats(gram1_ref, sx_ref, w1_ref[...],
                           g1_ref[...], be1_ref[...], count)
    w = w2_ref[...].reshape(3 * cmid, cmid).astype(_BF16)
    w1s = (w1_ref[...] * sc1).astype(_BF16)
    y1 = jnp.dot(xr_ref[...], w1s,
                 preferred_element_type=_F32)
    h = _leaky(y1 + sh1)
    hb = h.astype(_BF16)
    tm, c = hb.shape
    z = jnp.zeros((1, c), _BF16)
    left = jnp.concatenate([z, hb[:tm - 1, :]], axis=0)
    right = jnp.concatenate([hb[1:, :], z], axis=0)
    pos = jax.lax.broadcasted_iota(jnp.int32, (tm, 1), 0) % L
    left = jnp.where(pos == 0, z, left)
    right = jnp.where(pos == L - 1, z, right)
    hin = jnp.concatenate([left, hb, right], axis=1)
    y = jnp.dot(hin, w, preferred_element_type=_F32)
    h2_ref[...] = y.astype(_BF16)
    s_ref[0] = jnp.sum(y, axis=0, keepdims=True)
    q_ref[0] = jnp.sum(y * y, axis=0, keepdims=True)


def _p3_kernel(h2_ref, s2_ref, q2_ref, g2_ref, be2_ref,
               gram_ref, s_ref, *, count):
    sc2, sh2 = _scale_shift(s2_ref, q2_ref, g2_ref[...], be2_ref[...], count)
    h = _leaky(h2_ref[...].astype(_F32) * sc2 + sh2)
    hb = h.astype(_BF16)
    gram_ref[0] = jax.lax.dot_general(hb, hb, (((0,), (0,)), ((), ())),
                                      preferred_element_type=_F32)
    s_ref[0] = jnp.sum(hb.astype(_F32), axis=0, keepdims=True)


def _p4_kernel(h2_ref, xr_ref, s2_ref, q2_ref, g2_ref, be2_ref,
               gram1_ref, sx_ref, gram3_ref, s3_ref,
               gid_ref, beid_ref, g3_ref, be3_ref,
               w3_ref, wid_ref, o_ref, *, spt, L, cmid, count):
    sc2, sh2 = _scale_shift(s2_ref, q2_ref, g2_ref[...], be2_ref[...], count)
    scid, shid = _gram_stats(gram1_ref, sx_ref, wid_ref[...],
                             gid_ref[...], beid_ref[...], count)
    sc3, sh3 = _gram_stats(gram3_ref, s3_ref, w3_ref[...],
                           g3_ref[...], be3_ref[...], count)
    c4 = w3_ref.shape[1]
    zrows = jnp.zeros((cmid - 1, c4), _BF16)
    w3a = jnp.concatenate(
        [(w3_ref[...] * sc3).astype(_BF16), sh3.astype(_BF16), zrows], axis=0)
    wida = jnp.concatenate(
        [(wid_ref[...] * scid).astype(_BF16), shid.astype(_BF16), zrows], axis=0)
    ones = jnp.ones((L, cmid), _BF16)
    for i in range(spt):
        hbi = _leaky(h2_ref[i * L:(i + 1) * L, :].astype(_F32) * sc2
                     + sh2).astype(_BF16)
        hx = jnp.concatenate([hbi, ones], axis=1)
        a = _leaky(jnp.dot(hx, w3a, preferred_element_type=_F32))
        xo = jnp.concatenate([xr_ref[i * L:(i + 1) * L, :], ones], axis=1)
        idn = jnp.dot(xo, wida, preferred_element_type=_F32)
        o = _leaky(a + idn)
        o_ref[i] = jnp.transpose(o)


def kernel(x_ncl, w1, b1, g1, be1, w2, b2, g2, be2, w3, b3, g3, be3,
           wid, bid, gid, beid):
    N, Cin, L = x_ncl.shape
    Cmid = w1.shape[1]
    C4 = w3.shape[1]
    NL = N * L

    spt = 128
    while N % spt:
        spt //= 2
    T = N // spt
    TM = spt * L

    cparams = pltpu.CompilerParams(
        dimension_semantics=("parallel",),
        vmem_limit_bytes=64 * 1024 * 1024,
    )
    row_spec = lambda c: pl.BlockSpec((TM, c), lambda i: (i, 0))
    vec_spec = lambda c: pl.BlockSpec((1, c), lambda i: (0, 0))
    full_spec = lambda r, c: pl.BlockSpec((r, c), lambda i: (0, 0))
    ncl_spec = lambda c: pl.BlockSpec((spt, c, L), lambda i: (i, 0, 0))
    stat_spec = lambda c: pl.BlockSpec((1, 1, c), lambda i: (i, 0, 0))
    allstat_spec = lambda c: pl.BlockSpec((T, 1, c), lambda i: (0, 0, 0))
    stat_shape = lambda c: jax.ShapeDtypeStruct((T, 1, c), _F32)
    slab_shape = lambda c: jax.ShapeDtypeStruct((NL, c), _BF16)
    w2_spec = pl.BlockSpec((3, Cmid, Cmid), lambda i: (0, 0, 0))

    gram_out_spec = pl.BlockSpec((1, Cin, Cin), lambda i: (i, 0, 0))
    gram_in_spec = pl.BlockSpec((T, Cin, Cin), lambda i: (0, 0, 0))
    gram_shape = jax.ShapeDtypeStruct((T, Cin, Cin), _F32)

    xr, gram1, sx = pl.pallas_call(
        functools.partial(_p1_kernel, spt=spt, L=L),
        grid=(T,),
        in_specs=[ncl_spec(Cin)],
        out_specs=[row_spec(Cin), gram_out_spec, stat_spec(Cin)],
        out_shape=[slab_shape(Cin), gram_shape, stat_shape(Cin)],
        compiler_params=cparams,
    )(x_ncl)

    h2, s2, q2 = pl.pallas_call(
        functools.partial(_p2_kernel, L=L, cmid=Cmid, count=float(NL)),
        grid=(T,),
        in_specs=[row_spec(Cin), full_spec(Cin, Cmid),
                  gram_in_spec, allstat_spec(Cin),
                  vec_spec(Cmid), vec_spec(Cmid), w2_spec],
        out_specs=[row_spec(Cmid), stat_spec(Cmid), stat_spec(Cmid)],
        out_shape=[slab_shape(Cmid), stat_shape(Cmid), stat_shape(Cmid)],
        compiler_params=cparams,
    )(xr, w1, gram1, sx, g1, be1, w2)

    gram3, s3 = pl.pallas_call(
        functools.partial(_p3_kernel, count=float(NL)),
        grid=(T,),
        in_specs=[row_spec(Cmid), allstat_spec(Cmid), allstat_spec(Cmid),
                  vec_spec(Cmid), vec_spec(Cmid)],
        out_specs=[pl.BlockSpec((1, Cmid, Cmid), lambda i: (i, 0, 0)),
                   stat_spec(Cmid)],
        out_shape=[jax.ShapeDtypeStruct((T, Cmid, Cmid), _F32),
                   stat_shape(Cmid)],
        compiler_params=cparams,
    )(h2, s2, q2, g2, be2)

    spt4 = min(spt, 64)
    T4 = N // spt4
    TM4 = spt4 * L
    row4_spec = lambda c: pl.BlockSpec((TM4, c), lambda i: (i, 0))
    out = pl.pallas_call(
        functools.partial(_p4_kernel, spt=spt4, L=L, cmid=Cmid, count=float(NL)),
        grid=(T4,),
        in_specs=[row4_spec(Cmid), row4_spec(Cin),
                  allstat_spec(Cmid), allstat_spec(Cmid),
                  vec_spec(Cmid), vec_spec(Cmid),
                  gram_in_spec, allstat_spec(Cin),
                  pl.BlockSpec((T, Cmid, Cmid), lambda i: (0, 0, 0)),
                  allstat_spec(Cmid),
                  vec_spec(C4), vec_spec(C4),
                  vec_spec(C4), vec_spec(C4),
                  full_spec(Cmid, C4), full_spec(Cin, C4)],
        out_specs=pl.BlockSpec((spt4, C4, L), lambda i: (i, 0, 0)),
        out_shape=jax.ShapeDtypeStruct((N, C4, L), _F32),
        compiler_params=cparams,
    )(h2, xr, s2, q2, g2, be2, gram1, sx, gram3, s3,
      gid, beid, g3, be3, w3, wid)
    return out
